# baseline (device time: 365705 ns/iter reference)
import jax
import jax.numpy as jnp
from jax import lax
from jax.experimental import pallas as pl
from jax.experimental.pallas import tpu as pltpu

N_DEV = 8
SEQ = 1024
HEADS = 8
HH = HEADS // 2
DH = 128
HCOL = HH * DH
NGRP = 4
GRP = SEQ // NGRP
SCALE = 0.08838834764831843

_ORDER = [0, 4, 8, 12, 1, 5, 9, 13, 2, 6, 10, 14, 3, 7, 11, 15]


def _body(x_any, wq_ref, wo_ref, k_any, v_any, out_any,
          commQA, commOA, commQB, commOB,
          kbufA, vbufA, kbufB, vbufB,
          xb_ref, q_ref, ctx_ref, acc_ref,
          send_sems, recv_sems, kv_sems, io_sem, creditA, creditB):
    i = lax.axis_index("i")
    left = lax.rem(i + N_DEV - 1, N_DEV)
    right = lax.rem(i + 1, N_DEV)

    def kv_copies(h, slot):
        jA = lax.rem(i - h + 2 * N_DEV, N_DEV)
        jB = lax.rem(i + h, N_DEV)
        out = []
        for g in range(16):
            src_rows = pl.ds(_ORDER[g] * 64, 64)
            dst_rows = pl.ds(g * 64, 64)
            for s, (buf, src, joff) in enumerate((
                (kbufA, k_any, jA * HEADS),
                (vbufA, v_any, jA * HEADS),
                (kbufB, k_any, jB * HEADS + HH),
                (vbufB, v_any, jB * HEADS + HH),
            )):
                out.append(pltpu.make_async_copy(
                    src.at[i, src_rows, pl.ds(joff, HH), :],
                    buf.at[slot, dst_rows, :, :],
                    kv_sems.at[slot, s],
                ))
        return out

    io_pending = [
        pltpu.make_async_copy(
            x_any.at[0, pl.ds(_ORDER[g] * 64, 64), :],
            xb_ref.at[pl.ds(g * 64, 64), :],
            io_sem,
        )
        for g in range(16)
    ] + [
        pltpu.make_async_copy(wq_ref.at[:, pl.ds(0, HCOL)],
                              commQA.at[1], io_sem),
        pltpu.make_async_copy(wq_ref.at[:, pl.ds(HCOL, HCOL)],
                              commQB.at[1], io_sem),
        pltpu.make_async_copy(wo_ref.at[pl.ds(0, HCOL), :],
                              commOA.at[1], io_sem),
        pltpu.make_async_copy(wo_ref.at[pl.ds(HCOL, HCOL), :],
                              commOB.at[1], io_sem),
    ]
    for c in io_pending:
        c.start()
    for c in kv_copies(0, 0):
        c.start()

    barrier = pltpu.get_barrier_semaphore()
    for nbr in (left, right):
        pl.semaphore_signal(barrier, inc=1, device_id=(nbr,),
                            device_id_type=pl.DeviceIdType.MESH)
    pl.semaphore_wait(barrier, 2)

    for c in io_pending:
        c.wait()

    acc_ref[...] = jnp.zeros((SEQ, SEQ), jnp.float32)

    def hop(h, carry):
        slot = lax.rem(h, 2)
        prev = lax.rem(h + 1, 2)
        srcQA = commQA.at[prev]
        srcQB = commQB.at[prev]
        srcOA = commOA.at[prev]
        srcOB = commOB.at[prev]

        rdmas = [
            pltpu.make_async_remote_copy(
                src_ref=src,
                dst_ref=dst.at[slot],
                send_sem=send_sems.at[s],
                recv_sem=recv_sems.at[s],
                device_id=(tgt,),
                device_id_type=pl.DeviceIdType.MESH,
            )
            for s, (src, dst, tgt) in enumerate((
                (srcQA, commQA, right),
                (srcOA, commOA, right),
                (srcQB, commQB, left),
                (srcOB, commOB, left),
            ))
        ]

        @pl.when(jnp.logical_and(h >= 1, h < N_DEV - 1))
        def _():
            pl.semaphore_wait(creditA, 1)
            pl.semaphore_wait(creditB, 1)

        @pl.when(h < N_DEV - 1)
        def _():
            for r in rdmas:
                r.start()
            for c in kv_copies(h + 1, prev):
                c.start()

        q_ref[:, pl.ds(0, HCOL)] = jnp.dot(
            xb_ref[...], srcQA[...], preferred_element_type=jnp.float32)
        q_ref[:, pl.ds(HCOL, HCOL)] = jnp.dot(
            xb_ref[...], srcQB[...], preferred_element_type=jnp.float32)

        for c in kv_copies(h, slot):
            c.wait()

        def attn_group(r, _):
            rows = pl.ds(r * GRP, GRP)
            for kb, vb, coff in ((kbufA, vbufA, 0), (kbufB, vbufB, HCOL)):
                for hh in range(HH):
                    cols = pl.ds(coff + hh * DH, DH)
                    q = q_ref[rows, cols]
                    k = kb[slot, rows, hh, :]
                    s = lax.dot_general(
                        q, k, (((1,), (1,)), ((), ())),
                        preferred_element_type=jnp.float32) * SCALE
                    m = jnp.max(s, axis=1, keepdims=True)
                    e = jnp.exp(s - m)
                    den = jnp.sum(e, axis=1, keepdims=True)
                    v = vb[slot, rows, hh, :]
                    ctx_ref[rows, cols] = jnp.dot(
                        e, v, preferred_element_type=jnp.float32) / den
            return 0

        lax.fori_loop(0, NGRP, attn_group, 0)

        contrib = jnp.dot(ctx_ref[:, pl.ds(0, HCOL)], srcOA[...],
                          preferred_element_type=jnp.float32)
        contrib = contrib + jnp.dot(ctx_ref[:, pl.ds(HCOL, HCOL)], srcOB[...],
                                    preferred_element_type=jnp.float32)
        acc_ref[...] = acc_ref[...] + contrib

        @pl.when(h < N_DEV - 1)
        def _():
            for r in rdmas:
                r.wait()

        @pl.when(h < N_DEV - 2)
        def _():
            pl.semaphore_signal(creditA, inc=1, device_id=(left,),
                                device_id_type=pl.DeviceIdType.MESH)
            pl.semaphore_signal(creditB, inc=1, device_id=(right,),
                                device_id_type=pl.DeviceIdType.MESH)
        return carry

    lax.fori_loop(0, N_DEV, hop, 0)

    out_copies = [
        pltpu.make_async_copy(
            acc_ref.at[pl.ds(g * 64, 64), :],
            out_any.at[0, pl.ds(_ORDER[g] * 64, 64), :],
            io_sem,
        )
        for g in range(16)
    ]
    for c in out_copies:
        c.start()
    for c in out_copies:
        c.wait()


def kernel(x, Wq, K_ext, V_ext, Wo):
    return pl.pallas_call(
        _body,
        out_shape=jax.ShapeDtypeStruct((1, SEQ, SEQ), jnp.float32),
        in_specs=[
            pl.BlockSpec(memory_space=pl.ANY),
            pl.BlockSpec(memory_space=pltpu.MemorySpace.VMEM),
            pl.BlockSpec(memory_space=pltpu.MemorySpace.VMEM),
            pl.BlockSpec(memory_space=pl.ANY),
            pl.BlockSpec(memory_space=pl.ANY),
        ],
        out_specs=pl.BlockSpec(memory_space=pl.ANY),
        scratch_shapes=[
            pltpu.VMEM((2, SEQ, HCOL), jnp.float32),
            pltpu.VMEM((2, HCOL, SEQ), jnp.float32),
            pltpu.VMEM((2, SEQ, HCOL), jnp.float32),
            pltpu.VMEM((2, HCOL, SEQ), jnp.float32),
            pltpu.VMEM((2, SEQ, HH, DH), jnp.float32),
            pltpu.VMEM((2, SEQ, HH, DH), jnp.float32),
            pltpu.VMEM((2, SEQ, HH, DH), jnp.float32),
            pltpu.VMEM((2, SEQ, HH, DH), jnp.float32),
            pltpu.VMEM((SEQ, SEQ), jnp.float32),
            pltpu.VMEM((SEQ, SEQ), jnp.float32),
            pltpu.VMEM((SEQ, SEQ), jnp.float32),
            pltpu.VMEM((SEQ, SEQ), jnp.float32),
            pltpu.SemaphoreType.DMA((4,)),
            pltpu.SemaphoreType.DMA((4,)),
            pltpu.SemaphoreType.DMA((2, 4)),
            pltpu.SemaphoreType.DMA,
            pltpu.SemaphoreType.REGULAR,
            pltpu.SemaphoreType.REGULAR,
        ],
        compiler_params=pltpu.CompilerParams(
            collective_id=0, vmem_limit_bytes=100 * 1024 * 1024),
    )(x, Wq, Wo, K_ext, V_ext)
